# baseline (speedup 1.0000x reference)
import functools

import jax
import jax.numpy as jnp
from jax import lax
from jax.experimental import pallas as pl
from jax.experimental.pallas import tpu as pltpu
from jax.experimental.pallas import tpu_sc as plsc

_HID = 128
_CD = 16
_NORB = 16

_NC = 2
_NS = 16
_NW = _NC * _NS
_CHUNK = 128
_RING = 5

_B_TC = 400


def _sc_gather(table, z_pad):
    b_pad = z_pad.shape[0]
    b_per_w = b_pad // _NW
    n_chunks = b_per_w // _CHUNK
    n_outer = n_chunks // _RING
    mesh = plsc.VectorSubcoreMesh(
        core_axis_name="c", subcore_axis_name="s",
        num_cores=_NC, num_subcores=_NS)

    n_rows_w = b_per_w // _CHUNK
    win = n_rows_w + 7

    def body(table_hbm, idx_hbm, out_hbm, idx_all, rows, gsem, ssem):
        wid = lax.axis_index("s") * _NC + lax.axis_index("c")
        base = wid * b_per_w
        start8 = (wid * n_rows_w) // 8 * 8
        delta = wid * n_rows_w - start8
        pltpu.sync_copy(idx_hbm.at[pl.ds(start8, win)], idx_all)

        def gath(j, b):
            return pltpu.make_async_copy(
                table_hbm.at[idx_all.at[delta + j]], rows[b], gsem[b])

        def scat(j, b):
            return pltpu.make_async_copy(
                rows[b], out_hbm.at[pl.ds(base + j * _CHUNK, _CHUNK)],
                ssem[b])

        for b in range(_RING):
            gath(b, b).start()

        def step(p, carry):
            for b in range(_RING):
                j = p * _RING + b
                gath(j, b).wait()
                scat(j, b).start()

                @pl.when(p < n_outer - 1)
                def _():
                    scat(j, b).wait()
                    gath(j + _RING, b).start()

            return carry

        lax.fori_loop(0, n_outer, step, 0)
        for b in range(_RING):
            scat(n_chunks - _RING + b, b).wait()

    f = pl.kernel(
        body,
        out_type=jax.ShapeDtypeStruct((b_pad, _HID), jnp.float32),
        mesh=mesh,
        scratch_types=[
            pltpu.VMEM((win, _CHUNK), jnp.int32),
            tuple(pltpu.VMEM((_CHUNK, _HID), jnp.float32)
                  for _ in range(_RING)),
            tuple(pltpu.SemaphoreType.DMA for _ in range(_RING)),
            tuple(pltpu.SemaphoreType.DMA for _ in range(_RING)),
        ],
    )
    return f(table, z_pad.reshape(b_pad // _CHUNK, _CHUNK))


def _silu(x):
    u = x * 0.5
    return u + u * jnp.tanh(u)


def _tc_body(cf_ref, ze_ref, w1_ref, w2_ref, out_ref):
    c = _silu(cf_ref[:].reshape(_B_TC * _NORB, _CD))
    h = jnp.dot(c, w1_ref[:], preferred_element_type=jnp.float32)
    h = _silu(h)
    s = jnp.sum(h.reshape(_B_TC, _NORB, _HID), axis=1)
    out_ref[:] = ze_ref[:] + jnp.dot(
        s, w2_ref[:], preferred_element_type=jnp.float32)


def _tc_mlp(ze_pad, coeffs, W1, W2, n):
    nblk = n // _B_TC
    return pl.pallas_call(
        _tc_body,
        grid=(nblk,),
        in_specs=[
            pl.BlockSpec((_B_TC, _NORB, _CD), lambda i: (i, 0, 0)),
            pl.BlockSpec((_B_TC, _HID), lambda i: (i, 0)),
            pl.BlockSpec((_CD, _HID), lambda i: (0, 0)),
            pl.BlockSpec((_HID, _HID), lambda i: (0, 0)),
        ],
        out_specs=pl.BlockSpec((_B_TC, _HID), lambda i: (i, 0)),
        out_shape=jax.ShapeDtypeStruct((n, _HID), jnp.float32),
    )(coeffs, ze_pad, W1, W2)


def kernel(z, coeffs, table, W1, W2):
    n = z.shape[0]
    granule = _NW * _CHUNK
    b_pad = ((n + granule - 1) // granule) * granule
    z_pad = jnp.pad(z.astype(jnp.int32), (0, b_pad - n))
    trows = table.shape[0]
    trows_pad = ((trows + 7) // 8) * 8
    table_pad = jnp.pad(table, ((0, trows_pad - trows), (0, 0)))
    ze_pad = _sc_gather(table_pad, z_pad)
    return _tc_mlp(ze_pad, coeffs, W1, W2, n)


# speedup vs baseline: 1.8466x; 1.8466x over previous
import functools

import jax
import jax.numpy as jnp
from jax import lax
from jax.experimental import pallas as pl
from jax.experimental.pallas import tpu as pltpu
from jax.experimental.pallas import tpu_sc as plsc

_HID = 128
_CD = 16
_NORB = 16

_NC = 2
_NS = 16
_NW = _NC * _NS
_CHUNK = 128
_RING = 5

_B_TC = 400


def _sc_gather(table, z_pad):
    b_pad = z_pad.shape[0]
    b_per_w = b_pad // _NW
    n_chunks = b_per_w // _CHUNK
    n_outer = n_chunks // _RING
    mesh = plsc.VectorSubcoreMesh(
        core_axis_name="c", subcore_axis_name="s",
        num_cores=_NC, num_subcores=_NS)

    n_rows_w = b_per_w // _CHUNK
    win = n_rows_w + 7

    def body(table_hbm, idx_hbm, out_hbm, idx_all, rows, gsem, ssem):
        wid = lax.axis_index("s") * _NC + lax.axis_index("c")
        base = wid * b_per_w
        start8 = (wid * n_rows_w) // 8 * 8
        delta = wid * n_rows_w - start8
        pltpu.sync_copy(idx_hbm.at[pl.ds(start8, win)], idx_all)

        def gath(j, b):
            return pltpu.make_async_copy(
                table_hbm.at[idx_all.at[delta + j]], rows[b], gsem[b])

        def scat(j, b):
            return pltpu.make_async_copy(
                rows[b], out_hbm.at[pl.ds(base + j * _CHUNK, _CHUNK)],
                ssem[b])

        for b in range(_RING):
            gath(b, b).start()

        def step(p, carry):
            for b in range(_RING):
                j = p * _RING + b
                gath(j, b).wait()
                scat(j, b).start()

                @pl.when(p < n_outer - 1)
                def _():
                    scat(j, b).wait()
                    gath(j + _RING, b).start()

            return carry

        lax.fori_loop(0, n_outer, step, 0)
        for b in range(_RING):
            scat(n_chunks - _RING + b, b).wait()

    f = pl.kernel(
        body,
        out_type=jax.ShapeDtypeStruct((b_pad, _HID), jnp.float32),
        mesh=mesh,
        scratch_types=[
            pltpu.VMEM((win, _CHUNK), jnp.int32),
            tuple(pltpu.VMEM((_CHUNK, _HID), jnp.float32)
                  for _ in range(_RING)),
            tuple(pltpu.SemaphoreType.DMA for _ in range(_RING)),
            tuple(pltpu.SemaphoreType.DMA for _ in range(_RING)),
        ],
    )
    return f(table, z_pad.reshape(b_pad // _CHUNK, _CHUNK))


def _silu(x):
    u = x * 0.5
    return u + u * jnp.tanh(u)


def _tc_body(cf_ref, ze_ref, w1bd_ref, w2_ref, out_ref):
    cs = _silu(cf_ref[:])
    h_lo = _silu(jnp.dot(cs[:, :_HID], w1bd_ref[:],
                         preferred_element_type=jnp.float32))
    h_hi = _silu(jnp.dot(cs[:, _HID:], w1bd_ref[:],
                         preferred_element_type=jnp.float32))
    s = h_lo[:, :_HID] + h_hi[:, :_HID]
    for o in range(1, _NORB // 2):
        s = s + h_lo[:, o * _HID:(o + 1) * _HID]
        s = s + h_hi[:, o * _HID:(o + 1) * _HID]
    out_ref[:] = ze_ref[:] + jnp.dot(
        s, w2_ref[:], preferred_element_type=jnp.float32)


def _tc_mlp(ze_pad, coeffs256, W1bd, W2, n):
    nblk = n // _B_TC
    ngrp = _NORB // 2
    return pl.pallas_call(
        _tc_body,
        grid=(nblk,),
        in_specs=[
            pl.BlockSpec((_B_TC, _NORB * _CD), lambda i: (i, 0)),
            pl.BlockSpec((_B_TC, _HID), lambda i: (i, 0)),
            pl.BlockSpec((ngrp * _CD, ngrp * _HID), lambda i: (0, 0)),
            pl.BlockSpec((_HID, _HID), lambda i: (0, 0)),
        ],
        out_specs=pl.BlockSpec((_B_TC, _HID), lambda i: (i, 0)),
        out_shape=jax.ShapeDtypeStruct((n, _HID), jnp.float32),
    )(coeffs256, ze_pad, W1bd, W2)


def kernel(z, coeffs, table, W1, W2):
    n = z.shape[0]
    granule = _NW * _CHUNK
    b_pad = ((n + granule - 1) // granule) * granule
    z_pad = jnp.pad(z.astype(jnp.int32), (0, b_pad - n))
    trows = table.shape[0]
    trows_pad = ((trows + 7) // 8) * 8
    table_pad = jnp.pad(table, ((0, trows_pad - trows), (0, 0)))
    ze_pad = _sc_gather(table_pad, z_pad)
    coeffs256 = coeffs.reshape(n, _NORB * _CD)
    ngrp = _NORB // 2
    eye = jnp.eye(ngrp, dtype=W1.dtype)
    w1bd = (eye[:, None, :, None] * W1[None, :, None, :]).reshape(
        ngrp * _CD, ngrp * _HID)
    return _tc_mlp(ze_pad, coeffs256, w1bd, W2, n)


# speedup vs baseline: 2.2114x; 1.1976x over previous
import functools

import jax
import jax.numpy as jnp
from jax import lax
from jax.experimental import pallas as pl
from jax.experimental.pallas import tpu as pltpu
from jax.experimental.pallas import tpu_sc as plsc

_HID = 128
_CD = 16
_NORB = 16

_NC = 2
_NS = 16
_NW = _NC * _NS
_CHUNK = 128
_RING = 5

_B_TC = 512


def _sc_gather(table, z_pad):
    b_pad = z_pad.shape[0]
    b_per_w = b_pad // _NW
    n_chunks = b_per_w // _CHUNK
    n_outer = n_chunks // _RING
    mesh = plsc.VectorSubcoreMesh(
        core_axis_name="c", subcore_axis_name="s",
        num_cores=_NC, num_subcores=_NS)

    n_rows_w = b_per_w // _CHUNK
    win = n_rows_w + 7

    def body(table_hbm, idx_hbm, out_hbm, idx_all, rows, gsem, ssem):
        wid = lax.axis_index("s") * _NC + lax.axis_index("c")
        base = wid * b_per_w
        start8 = (wid * n_rows_w) // 8 * 8
        delta = wid * n_rows_w - start8
        pltpu.sync_copy(idx_hbm.at[pl.ds(start8, win)], idx_all)

        def gath(j, b):
            return pltpu.make_async_copy(
                table_hbm.at[idx_all.at[delta + j]], rows[b], gsem[b])

        def scat(j, b):
            return pltpu.make_async_copy(
                rows[b], out_hbm.at[pl.ds(base + j * _CHUNK, _CHUNK)],
                ssem[b])

        for b in range(_RING):
            gath(b, b).start()

        def step(p, carry):
            for b in range(_RING):
                j = p * _RING + b
                gath(j, b).wait()
                scat(j, b).start()

                @pl.when(p < n_outer - 1)
                def _():
                    scat(j, b).wait()
                    gath(j + _RING, b).start()

            return carry

        lax.fori_loop(0, n_outer, step, 0)
        for b in range(_RING):
            scat(n_chunks - _RING + b, b).wait()

    f = pl.kernel(
        body,
        out_type=jax.ShapeDtypeStruct((b_pad, _HID), jnp.float32),
        mesh=mesh,
        scratch_types=[
            pltpu.VMEM((win, _CHUNK), jnp.int32),
            tuple(pltpu.VMEM((_CHUNK, _HID), jnp.float32)
                  for _ in range(_RING)),
            tuple(pltpu.SemaphoreType.DMA for _ in range(_RING)),
            tuple(pltpu.SemaphoreType.DMA for _ in range(_RING)),
        ],
    )
    return f(table, z_pad.reshape(b_pad // _CHUNK, _CHUNK))


def _silu(x):
    u = x * 0.5
    return u + u * jnp.tanh(u)


def _tc_body(cf_ref, ze_ref, w1bdt_ref, w2_ref, out_ref):
    cs = _silu(cf_ref[:])
    h_lo = _silu(jnp.dot(w1bdt_ref[:], cs[:_HID, :],
                         preferred_element_type=jnp.float32))
    h_hi = _silu(jnp.dot(w1bdt_ref[:], cs[_HID:, :],
                         preferred_element_type=jnp.float32))
    st = h_lo[:_HID, :] + h_hi[:_HID, :]
    for o in range(1, _NORB // 2):
        st = st + h_lo[o * _HID:(o + 1) * _HID, :]
        st = st + h_hi[o * _HID:(o + 1) * _HID, :]
    s = jnp.transpose(st)
    out_ref[:] = ze_ref[:] + jnp.dot(
        s, w2_ref[:], preferred_element_type=jnp.float32)


def _tc_mlp(ze_pad, coeffsT, W1bdT, W2, n):
    nblk = (n + _B_TC - 1) // _B_TC
    ngrp = _NORB // 2
    return pl.pallas_call(
        _tc_body,
        grid=(nblk,),
        in_specs=[
            pl.BlockSpec((_NORB * _CD, _B_TC), lambda i: (0, i)),
            pl.BlockSpec((_B_TC, _HID), lambda i: (i, 0)),
            pl.BlockSpec((ngrp * _HID, ngrp * _CD), lambda i: (0, 0)),
            pl.BlockSpec((_HID, _HID), lambda i: (0, 0)),
        ],
        out_specs=pl.BlockSpec((_B_TC, _HID), lambda i: (i, 0)),
        out_shape=jax.ShapeDtypeStruct((n, _HID), jnp.float32),
    )(coeffsT, ze_pad, W1bdT, W2)


def kernel(z, coeffs, table, W1, W2):
    n = z.shape[0]
    granule = _NW * _CHUNK
    b_pad = ((n + granule - 1) // granule) * granule
    z_pad = jnp.pad(z.astype(jnp.int32), (0, b_pad - n))
    trows = table.shape[0]
    trows_pad = ((trows + 7) // 8) * 8
    table_pad = jnp.pad(table, ((0, trows_pad - trows), (0, 0)))
    ze_pad = _sc_gather(table_pad, z_pad)
    coeffsT = coeffs.reshape(n, _NORB * _CD).T
    ngrp = _NORB // 2
    eye = jnp.eye(ngrp, dtype=W1.dtype)
    w1bdt = (eye[:, None, :, None] * W1.T[None, :, None, :]).reshape(
        ngrp * _HID, ngrp * _CD)
    return _tc_mlp(ze_pad, coeffsT, w1bdt, W2, n)


# speedup vs baseline: 2.4951x; 1.1283x over previous
import functools

import jax
import jax.numpy as jnp
from jax import lax
from jax.experimental import pallas as pl
from jax.experimental.pallas import tpu as pltpu
from jax.experimental.pallas import tpu_sc as plsc

_HID = 128
_CD = 16
_NORB = 16

_NC = 2
_NS = 16
_NW = _NC * _NS
_CHUNK = 128
_RING = 5

_B_TC = 512


def _sc_gather(table, z_pad):
    b_pad = z_pad.shape[0]
    b_per_w = b_pad // _NW
    n_chunks = b_per_w // _CHUNK
    n_outer = n_chunks // _RING
    mesh = plsc.VectorSubcoreMesh(
        core_axis_name="c", subcore_axis_name="s",
        num_cores=_NC, num_subcores=_NS)

    n_rows_w = b_per_w // _CHUNK
    win = n_rows_w + 7

    def body(table_hbm, idx_hbm, out_hbm, idx_all, rows, gsem, ssem):
        wid = lax.axis_index("s") * _NC + lax.axis_index("c")
        base = wid * b_per_w
        start8 = (wid * n_rows_w) // 8 * 8
        delta = wid * n_rows_w - start8
        pltpu.sync_copy(idx_hbm.at[pl.ds(start8, win)], idx_all)

        def gath(j, b):
            return pltpu.make_async_copy(
                table_hbm.at[idx_all.at[delta + j]], rows[b], gsem[b])

        def scat(j, b):
            return pltpu.make_async_copy(
                rows[b], out_hbm.at[pl.ds(base + j * _CHUNK, _CHUNK)],
                ssem[b])

        for b in range(_RING):
            gath(b, b).start()

        def step(p, carry):
            for b in range(_RING):
                j = p * _RING + b
                gath(j, b).wait()
                scat(j, b).start()

                @pl.when(p < n_outer - 1)
                def _():
                    scat(j, b).wait()
                    gath(j + _RING, b).start()

            return carry

        lax.fori_loop(0, n_outer, step, 0)
        for b in range(_RING):
            scat(n_chunks - _RING + b, b).wait()

    f = pl.kernel(
        body,
        out_type=jax.ShapeDtypeStruct((b_pad, _HID), jnp.float32),
        mesh=mesh,
        scratch_types=[
            pltpu.VMEM((win, _CHUNK), jnp.int32),
            tuple(pltpu.VMEM((_CHUNK, _HID), jnp.float32)
                  for _ in range(_RING)),
            tuple(pltpu.SemaphoreType.DMA for _ in range(_RING)),
            tuple(pltpu.SemaphoreType.DMA for _ in range(_RING)),
        ],
    )
    return f(table, z_pad.reshape(b_pad // _CHUNK, _CHUNK))


def _silu(x):
    u = x * 0.5
    return u + u * jnp.tanh(u)


def _tc_body(cf_ref, w1bdt_ref, w2_ref, out_ref):
    cs = _silu(cf_ref[:])
    h_lo = _silu(jnp.dot(w1bdt_ref[:], cs[:_HID, :],
                         preferred_element_type=jnp.float32))
    h_hi = _silu(jnp.dot(w1bdt_ref[:], cs[_HID:, :],
                         preferred_element_type=jnp.float32))
    st = h_lo[:_HID, :] + h_hi[:_HID, :]
    for o in range(1, _NORB // 2):
        st = st + h_lo[o * _HID:(o + 1) * _HID, :]
        st = st + h_hi[o * _HID:(o + 1) * _HID, :]
    s = jnp.transpose(st)
    out_ref[:] = jnp.dot(s, w2_ref[:], preferred_element_type=jnp.float32)


def _tc_mlp(coeffsT, W1bdT, W2, n):
    nblk = (n + _B_TC - 1) // _B_TC
    ngrp = _NORB // 2
    return pl.pallas_call(
        _tc_body,
        grid=(nblk,),
        in_specs=[
            pl.BlockSpec((_NORB * _CD, _B_TC), lambda i: (0, i)),
            pl.BlockSpec((ngrp * _HID, ngrp * _CD), lambda i: (0, 0)),
            pl.BlockSpec((_HID, _HID), lambda i: (0, 0)),
        ],
        out_specs=pl.BlockSpec((_B_TC, _HID), lambda i: (i, 0)),
        out_shape=jax.ShapeDtypeStruct((n, _HID), jnp.float32),
    )(coeffsT, W1bdT, W2)


def _add_body(a_ref, b_ref, out_ref):
    out_ref[:] = a_ref[:] + b_ref[:]


_B_ADD = 2000


def _tc_add(mlp, ze_pad, n):
    return pl.pallas_call(
        _add_body,
        grid=(n // _B_ADD,),
        in_specs=[
            pl.BlockSpec((_B_ADD, _HID), lambda i: (i, 0)),
            pl.BlockSpec((_B_ADD, _HID), lambda i: (i, 0)),
        ],
        out_specs=pl.BlockSpec((_B_ADD, _HID), lambda i: (i, 0)),
        out_shape=jax.ShapeDtypeStruct((n, _HID), jnp.float32),
    )(mlp, ze_pad)


def kernel(z, coeffs, table, W1, W2):
    n = z.shape[0]
    granule = _NW * _CHUNK
    b_pad = ((n + granule - 1) // granule) * granule
    z_pad = jnp.pad(z.astype(jnp.int32), (0, b_pad - n))
    trows = table.shape[0]
    trows_pad = ((trows + 7) // 8) * 8
    table_pad = jnp.pad(table, ((0, trows_pad - trows), (0, 0)))
    ze_pad = _sc_gather(table_pad, z_pad)
    coeffsT = coeffs.reshape(n, _NORB * _CD).T
    ngrp = _NORB // 2
    eye = jnp.eye(ngrp, dtype=W1.dtype)
    w1bdt = (eye[:, None, :, None] * W1.T[None, :, None, :]).reshape(
        ngrp * _HID, ngrp * _CD)
    mlp = _tc_mlp(coeffsT, w1bdt, W2, n)
    return _tc_add(mlp, ze_pad, n)


# speedup vs baseline: 2.8618x; 1.1470x over previous
import functools

import jax
import jax.numpy as jnp
from jax import lax
from jax.experimental import pallas as pl
from jax.experimental.pallas import tpu as pltpu
from jax.experimental.pallas import tpu_sc as plsc

_HID = 128
_CD = 16
_NORB = 16

_NC = 2
_NS = 16
_NW = _NC * _NS
_CHUNK = 128
_RING = 5

_B_TC = 1024


def _sc_gather(table, z_pad):
    b_pad = z_pad.shape[0]
    b_per_w = b_pad // _NW
    n_chunks = b_per_w // _CHUNK
    n_outer = n_chunks // _RING
    mesh = plsc.VectorSubcoreMesh(
        core_axis_name="c", subcore_axis_name="s",
        num_cores=_NC, num_subcores=_NS)

    n_rows_w = b_per_w // _CHUNK
    win = n_rows_w + 7

    def body(table_hbm, idx_hbm, out_hbm, idx_all, rows, gsem, ssem):
        wid = lax.axis_index("s") * _NC + lax.axis_index("c")
        base = wid * b_per_w
        start8 = (wid * n_rows_w) // 8 * 8
        delta = wid * n_rows_w - start8
        pltpu.sync_copy(idx_hbm.at[pl.ds(start8, win)], idx_all)

        def gath(j, b):
            return pltpu.make_async_copy(
                table_hbm.at[idx_all.at[delta + j]], rows[b], gsem[b])

        def scat(j, b):
            return pltpu.make_async_copy(
                rows[b], out_hbm.at[pl.ds(base + j * _CHUNK, _CHUNK)],
                ssem[b])

        for b in range(_RING):
            gath(b, b).start()

        def step(p, carry):
            for b in range(_RING):
                j = p * _RING + b
                gath(j, b).wait()
                scat(j, b).start()

                @pl.when(p < n_outer - 1)
                def _():
                    scat(j, b).wait()
                    gath(j + _RING, b).start()

            return carry

        lax.fori_loop(0, n_outer, step, 0)
        for b in range(_RING):
            scat(n_chunks - _RING + b, b).wait()

    f = pl.kernel(
        body,
        out_type=jax.ShapeDtypeStruct((b_pad, _HID), jnp.float32),
        mesh=mesh,
        scratch_types=[
            pltpu.VMEM((win, _CHUNK), jnp.int32),
            tuple(pltpu.VMEM((_CHUNK, _HID), jnp.float32)
                  for _ in range(_RING)),
            tuple(pltpu.SemaphoreType.DMA for _ in range(_RING)),
            tuple(pltpu.SemaphoreType.DMA for _ in range(_RING)),
        ],
    )
    return f(table, z_pad.reshape(b_pad // _CHUNK, _CHUNK))


def _silu(x):
    u = x * 0.5
    return u + u * jnp.tanh(u)


def _tc_body(cf_ref, w1bdt_ref, w2_ref, out_ref):
    cs = _silu(cf_ref[:])
    h_lo = _silu(jnp.dot(w1bdt_ref[:], cs[:_HID, :],
                         preferred_element_type=jnp.float32))
    h_hi = _silu(jnp.dot(w1bdt_ref[:], cs[_HID:, :],
                         preferred_element_type=jnp.float32))
    parts = [h_lo[o * _HID:(o + 1) * _HID, :] for o in range(_NORB // 2)]
    parts += [h_hi[o * _HID:(o + 1) * _HID, :] for o in range(_NORB // 2)]
    while len(parts) > 1:
        parts = [parts[i] + parts[i + 1] for i in range(0, len(parts), 2)]
    s = jnp.transpose(parts[0])
    out_ref[:] = jnp.dot(s, w2_ref[:], preferred_element_type=jnp.float32)


def _tc_mlp(coeffsT, W1bdT, W2, n):
    nblk = (n + _B_TC - 1) // _B_TC
    ngrp = _NORB // 2
    return pl.pallas_call(
        _tc_body,
        grid=(nblk,),
        in_specs=[
            pl.BlockSpec((_NORB * _CD, _B_TC), lambda i: (0, i)),
            pl.BlockSpec((ngrp * _HID, ngrp * _CD), lambda i: (0, 0)),
            pl.BlockSpec((_HID, _HID), lambda i: (0, 0)),
        ],
        out_specs=pl.BlockSpec((_B_TC, _HID), lambda i: (i, 0)),
        out_shape=jax.ShapeDtypeStruct((n, _HID), jnp.float32),
    )(coeffsT, W1bdT, W2)


def _add_body(a_ref, b_ref, out_ref):
    out_ref[:] = a_ref[:] + b_ref[:]


_B_ADD = 2000


def _tc_add(mlp, ze_pad, n):
    return pl.pallas_call(
        _add_body,
        grid=(n // _B_ADD,),
        in_specs=[
            pl.BlockSpec((_B_ADD, _HID), lambda i: (i, 0)),
            pl.BlockSpec((_B_ADD, _HID), lambda i: (i, 0)),
        ],
        out_specs=pl.BlockSpec((_B_ADD, _HID), lambda i: (i, 0)),
        out_shape=jax.ShapeDtypeStruct((n, _HID), jnp.float32),
    )(mlp, ze_pad)


def kernel(z, coeffs, table, W1, W2):
    n = z.shape[0]
    granule = _NW * _CHUNK
    b_pad = ((n + granule - 1) // granule) * granule
    z_pad = jnp.pad(z.astype(jnp.int32), (0, b_pad - n))
    trows = table.shape[0]
    trows_pad = ((trows + 7) // 8) * 8
    table_pad = jnp.pad(table, ((0, trows_pad - trows), (0, 0)))
    ze_pad = _sc_gather(table_pad, z_pad)
    coeffsT = coeffs.reshape(n, _NORB * _CD).T
    ngrp = _NORB // 2
    eye = jnp.eye(ngrp, dtype=W1.dtype)
    w1bdt = (eye[:, None, :, None] * W1.T[None, :, None, :]).reshape(
        ngrp * _HID, ngrp * _CD)
    mlp = _tc_mlp(coeffsT, w1bdt, W2, n)
    return _tc_add(mlp, ze_pad, n)


# speedup vs baseline: 3.0963x; 1.0819x over previous
import functools

import jax
import jax.numpy as jnp
from jax import lax
from jax.experimental import pallas as pl
from jax.experimental.pallas import tpu as pltpu
from jax.experimental.pallas import tpu_sc as plsc

_HID = 128
_CD = 16
_NORB = 16

_NC = 2
_NS = 16
_NW = _NC * _NS
_CHUNK = 128
_RING = 5

_B_TC = 2048


def _sc_gather(table, z_pad):
    b_pad = z_pad.shape[0]
    b_per_w = b_pad // _NW
    n_chunks = b_per_w // _CHUNK
    n_outer = n_chunks // _RING
    mesh = plsc.VectorSubcoreMesh(
        core_axis_name="c", subcore_axis_name="s",
        num_cores=_NC, num_subcores=_NS)

    n_rows_w = b_per_w // _CHUNK
    win = n_rows_w + 7

    def body(table_hbm, idx_hbm, out_hbm, idx_all, rows, gsem, ssem):
        wid = lax.axis_index("s") * _NC + lax.axis_index("c")
        base = wid * b_per_w
        start8 = (wid * n_rows_w) // 8 * 8
        delta = wid * n_rows_w - start8
        pltpu.sync_copy(idx_hbm.at[pl.ds(start8, win)], idx_all)

        def gath(j, b):
            return pltpu.make_async_copy(
                table_hbm.at[idx_all.at[delta + j]], rows[b], gsem[b])

        def scat(j, b):
            return pltpu.make_async_copy(
                rows[b], out_hbm.at[pl.ds(base + j * _CHUNK, _CHUNK)],
                ssem[b])

        for b in range(_RING):
            gath(b, b).start()

        def step(p, carry):
            for b in range(_RING):
                j = p * _RING + b
                gath(j, b).wait()
                scat(j, b).start()

                @pl.when(p < n_outer - 1)
                def _():
                    scat(j, b).wait()
                    gath(j + _RING, b).start()

            return carry

        lax.fori_loop(0, n_outer, step, 0)
        for b in range(_RING):
            scat(n_chunks - _RING + b, b).wait()

    f = pl.kernel(
        body,
        out_type=jax.ShapeDtypeStruct((b_pad, _HID), jnp.float32),
        mesh=mesh,
        scratch_types=[
            pltpu.VMEM((win, _CHUNK), jnp.int32),
            tuple(pltpu.VMEM((_CHUNK, _HID), jnp.float32)
                  for _ in range(_RING)),
            tuple(pltpu.SemaphoreType.DMA for _ in range(_RING)),
            tuple(pltpu.SemaphoreType.DMA for _ in range(_RING)),
        ],
    )
    return f(table, z_pad.reshape(b_pad // _CHUNK, _CHUNK))


def _silu(x):
    u = x * 0.5
    return u + u * jnp.tanh(u)


def _tc_body(cf_ref, w1bdt_ref, w2_ref, out_ref):
    cs = _silu(cf_ref[:])
    h_lo = _silu(jnp.dot(w1bdt_ref[:], cs[:_HID, :],
                         preferred_element_type=jnp.float32))
    h_hi = _silu(jnp.dot(w1bdt_ref[:], cs[_HID:, :],
                         preferred_element_type=jnp.float32))
    parts = [h_lo[o * _HID:(o + 1) * _HID, :] for o in range(_NORB // 2)]
    parts += [h_hi[o * _HID:(o + 1) * _HID, :] for o in range(_NORB // 2)]
    while len(parts) > 1:
        parts = [parts[i] + parts[i + 1] for i in range(0, len(parts), 2)]
    s = jnp.transpose(parts[0])
    out_ref[:] = jnp.dot(s, w2_ref[:], preferred_element_type=jnp.float32)


def _tc_mlp(coeffsT, W1bdT, W2, n):
    nblk = (n + _B_TC - 1) // _B_TC
    ngrp = _NORB // 2
    return pl.pallas_call(
        _tc_body,
        grid=(nblk,),
        in_specs=[
            pl.BlockSpec((_NORB * _CD, _B_TC), lambda i: (0, i)),
            pl.BlockSpec((ngrp * _HID, ngrp * _CD), lambda i: (0, 0)),
            pl.BlockSpec((_HID, _HID), lambda i: (0, 0)),
        ],
        out_specs=pl.BlockSpec((_B_TC, _HID), lambda i: (i, 0)),
        out_shape=jax.ShapeDtypeStruct((n, _HID), jnp.float32),
    )(coeffsT, W1bdT, W2)


def _add_body(a_ref, b_ref, out_ref):
    out_ref[:] = a_ref[:] + b_ref[:]


_B_ADD = 2000


def _tc_add(mlp, ze_pad, n):
    return pl.pallas_call(
        _add_body,
        grid=(n // _B_ADD,),
        in_specs=[
            pl.BlockSpec((_B_ADD, _HID), lambda i: (i, 0)),
            pl.BlockSpec((_B_ADD, _HID), lambda i: (i, 0)),
        ],
        out_specs=pl.BlockSpec((_B_ADD, _HID), lambda i: (i, 0)),
        out_shape=jax.ShapeDtypeStruct((n, _HID), jnp.float32),
    )(mlp, ze_pad)


def kernel(z, coeffs, table, W1, W2):
    n = z.shape[0]
    granule = _NW * _CHUNK
    b_pad = ((n + granule - 1) // granule) * granule
    z_pad = jnp.pad(z.astype(jnp.int32), (0, b_pad - n))
    trows = table.shape[0]
    trows_pad = ((trows + 7) // 8) * 8
    table_pad = jnp.pad(table, ((0, trows_pad - trows), (0, 0)))
    ze_pad = _sc_gather(table_pad, z_pad)
    coeffsT = coeffs.reshape(n, _NORB * _CD).T
    ngrp = _NORB // 2
    eye = jnp.eye(ngrp, dtype=W1.dtype)
    w1bdt = (eye[:, None, :, None] * W1.T[None, :, None, :]).reshape(
        ngrp * _HID, ngrp * _CD)
    mlp = _tc_mlp(coeffsT, w1bdt, W2, n)
    return _tc_add(mlp, ze_pad, n)
